# baseline (speedup 1.0000x reference)
import functools

import jax
import jax.numpy as jnp
from jax import lax
from jax.experimental import pallas as pl
from jax.experimental.pallas import tpu as pltpu
from jax.experimental.pallas import tpu_sc as plsc

B = 16384
E = 64
L = 16
NC = 2
NS = 16
NW = NC * NS
BPW = B // NW

_MESH = dict(core_axis_name="c", subcore_axis_name="s")
_PARAMS = pltpu.CompilerParams(use_tc_tiling_on_sc=False)


def _sc_user_gather(uidx, user_emb, ubias):
    @functools.partial(
        pl.kernel,
        out_type=[
            jax.ShapeDtypeStruct((B, E), jnp.float32),
            jax.ShapeDtypeStruct((B,), jnp.float32),
        ],
        mesh=plsc.VectorSubcoreMesh(**_MESH),
        compiler_params=_PARAMS,
        scratch_types=[
            pltpu.VMEM((BPW,), jnp.int32),
            pltpu.VMEM((BPW, E), jnp.float32),
            pltpu.VMEM((BPW,), jnp.float32),
            pltpu.SemaphoreType.DMA,
            pltpu.SemaphoreType.DMA,
        ],
    )
    def k(uidx_hbm, uemb_hbm, ubias_hbm, urows_out, ub_out,
          uidx_v, urows_v, ub_v, sem0, sem1):
        wid = lax.axis_index("s") * NC + lax.axis_index("c")
        base = wid * BPW
        pltpu.sync_copy(uidx_hbm.at[pl.ds(base, BPW)], uidx_v)
        cp0 = pltpu.async_copy(uemb_hbm.at[uidx_v], urows_v, sem0)
        cp1 = pltpu.async_copy(ubias_hbm.at[uidx_v], ub_v, sem1)
        cp1.wait()
        pltpu.sync_copy(ub_v, ub_out.at[pl.ds(base, BPW)])
        cp0.wait()
        pltpu.sync_copy(urows_v, urows_out.at[pl.ds(base, BPW)])

    return k(uidx, user_emb, ubias)


def _sc_movie_dot(midx, movie_emb, mbias, urows, ubg):
    @functools.partial(
        pl.kernel,
        out_type=[
            jax.ShapeDtypeStruct((NW, L), jnp.float32),
            jax.ShapeDtypeStruct((B,), jnp.float32),
        ],
        mesh=plsc.VectorSubcoreMesh(**_MESH),
        compiler_params=_PARAMS,
        scratch_types=[
            pltpu.VMEM((BPW,), jnp.int32),
            pltpu.VMEM((BPW, E), jnp.float32),
            pltpu.VMEM((BPW, E), jnp.float32),
            pltpu.VMEM((BPW,), jnp.float32),
            pltpu.VMEM((BPW,), jnp.float32),
            pltpu.VMEM((BPW,), jnp.float32),
            pltpu.VMEM((L,), jnp.float32),
            pltpu.SemaphoreType.DMA,
            pltpu.SemaphoreType.DMA,
        ],
    )
    def k(midx_hbm, memb_hbm, mbias_hbm, urows_hbm, ubg_hbm,
          partials_hbm, bsum_hbm,
          midx_v, mrows_v, urows_v, mb_v, ubg_v, bs_v, acc_v, sem0, sem1):
        wid = lax.axis_index("s") * NC + lax.axis_index("c")
        base = wid * BPW
        pltpu.sync_copy(midx_hbm.at[pl.ds(base, BPW)], midx_v)
        cp0 = pltpu.async_copy(memb_hbm.at[midx_v], mrows_v, sem0)
        cp1 = pltpu.async_copy(mbias_hbm.at[midx_v], mb_v, sem1)
        pltpu.sync_copy(urows_hbm.at[pl.ds(base, BPW)], urows_v)
        pltpu.sync_copy(ubg_hbm.at[pl.ds(base, BPW)], ubg_v)
        cp1.wait()
        for c in range(BPW // L):
            bs_v[pl.ds(c * L, L)] = ubg_v[pl.ds(c * L, L)] + mb_v[pl.ds(c * L, L)]
        pltpu.sync_copy(bs_v, bsum_hbm.at[pl.ds(base, BPW)])
        cp0.wait()

        def body(i, acc):
            for j in range(E // L):
                acc = acc + urows_v[i, pl.ds(j * L, L)] * mrows_v[i, pl.ds(j * L, L)]
            return acc

        acc = lax.fori_loop(0, BPW, body, jnp.zeros((L,), jnp.float32))
        acc_v[...] = acc
        pltpu.sync_copy(acc_v, partials_hbm.at[wid])

    return k(midx, movie_emb, mbias, urows, ubg)


def _tc_finish(partials, bias2d):
    def body(p_ref, b_ref, o_ref):
        s = jnp.sum(p_ref[...])
        o_ref[...] = jax.nn.sigmoid(b_ref[...] + s)

    return pl.pallas_call(
        body,
        out_shape=jax.ShapeDtypeStruct(bias2d.shape, jnp.float32),
    )(partials, bias2d)


def kernel(inputs, user_emb, user_bias_tab, movie_emb, movie_bias_tab):
    uidx = inputs[:, 0]
    midx = inputs[:, 1]
    ubias = user_bias_tab[:, 0]
    mbias = movie_bias_tab[:, 0]
    urows, ubg = _sc_user_gather(uidx, user_emb, ubias)
    partials, bsum = _sc_movie_dot(midx, movie_emb, mbias, urows, ubg)
    out2d = _tc_finish(partials, bsum.reshape(128, 128))
    return out2d.reshape(B, 1)


# speedup vs baseline: 1.5151x; 1.5151x over previous
import functools

import jax
import jax.numpy as jnp
from jax import lax
from jax.experimental import pallas as pl
from jax.experimental.pallas import tpu as pltpu
from jax.experimental.pallas import tpu_sc as plsc

B = 16384
E = 64
V = 100000
L = 16
NC = 2
NS = 16
NW = NC * NS
VMAIN = 99968
VTAIL = V - VMAIN
VPAD = VMAIN + 128
S0 = 50048
S1 = VMAIN - S0
IQ = B // 4
T_U, T_M, T_UB, T_MB, T_LEN = 0, 2048, 4096, 4128, 5120


def _sc_main(uidx, midx, uembt, membt, ubt, mbt, tails):
    @functools.partial(
        pl.kernel,
        out_type=[
            jax.ShapeDtypeStruct((NW, 128), jnp.float32),
            jax.ShapeDtypeStruct((B,), jnp.float32),
            jax.ShapeDtypeStruct((B,), jnp.float32),
        ],
        mesh=plsc.VectorSubcoreMesh(core_axis_name="c", subcore_axis_name="s"),
        compiler_params=pltpu.CompilerParams(needs_layout_passes=False),
        scratch_types=[
            pltpu.VMEM((VPAD,), jnp.float32),
            pltpu.VMEM((B,), jnp.float32),
            pltpu.VMEM((IQ,), jnp.int32),
            pltpu.VMEM((T_LEN,), jnp.float32),
            pltpu.VMEM((128,), jnp.float32),
            pltpu.SemaphoreType.DMA,
            pltpu.SemaphoreType.DMA,
        ],
    )
    def k(uidx_hbm, midx_hbm, uembt_hbm, membt_hbm, ubt_hbm, mbt_hbm, tails_hbm,
          partials_out, bu_out, bm_out,
          rowv, uvec, idxv, tails_v, o128, sem0, sem1):
        wid = lax.axis_index("s") * NC + lax.axis_index("c")
        lanes = jnp.arange(L, dtype=jnp.int32)
        pltpu.sync_copy(tails_hbm, tails_v)

        def stage_row(src2d, row, tbase, is_bias):
            cp0 = pltpu.async_copy(src2d.at[row, pl.ds(0, S0)],
                                   rowv.at[pl.ds(0, S0)], sem0)
            cp1 = pltpu.async_copy(src2d.at[row, pl.ds(S0, S1)],
                                   rowv.at[pl.ds(S0, S1)], sem1)
            cp0.wait()
            cp1.wait()
            for c in range(VTAIL // L):
                rel = lanes + c * L
                fidx = tbase + (rel if is_bias else rel * E + row)
                rowv[pl.ds(VMAIN + c * L, L)] = plsc.load_gather(tails_v, [fidx])

        def extract_quarter(q):
            def body(i, _):
                for t in range(4):
                    off = i * E + t * L
                    ic = idxv[pl.ds(off, L)]
                    uvec[pl.ds(q * IQ + off, L)] = plsc.load_gather(rowv, [ic])
                return 0
            lax.fori_loop(0, IQ // E, body, 0)

        def accum_quarter(q, acc0):
            def body(i, acc):
                for t in range(4):
                    off = i * E + t * L
                    ic = idxv[pl.ds(off, L)]
                    g = plsc.load_gather(rowv, [ic])
                    acc = acc + g * uvec[pl.ds(q * IQ + off, L)]
                return acc
            return lax.fori_loop(0, IQ // E, body, acc0)

        def extract_all(idx_hbm):
            for q in range(4):
                pltpu.sync_copy(idx_hbm.at[pl.ds(q * IQ, IQ)], idxv)
                extract_quarter(q)

        @pl.when(wid == 0)
        def _():
            stage_row(ubt_hbm, 0, T_UB, True)
            extract_all(uidx_hbm)
            pltpu.sync_copy(uvec, bu_out)

        @pl.when(wid == 1)
        def _():
            stage_row(mbt_hbm, 0, T_MB, True)
            extract_all(midx_hbm)
            pltpu.sync_copy(uvec, bm_out)

        acc = jnp.zeros((L,), jnp.float32)
        for j in range(2):
            e = wid * 2 + j
            stage_row(uembt_hbm, e, T_U, False)
            extract_all(uidx_hbm)
            stage_row(membt_hbm, e, T_M, False)
            for q in range(4):
                pltpu.sync_copy(midx_hbm.at[pl.ds(q * IQ, IQ)], idxv)
                acc = accum_quarter(q, acc)

        o128[pl.ds(0, L)] = acc
        zeros = jnp.zeros((L,), jnp.float32)
        for c in range(1, 128 // L):
            o128[pl.ds(c * L, L)] = zeros
        pltpu.sync_copy(o128, partials_out.at[wid])

    return k(uidx, midx, uembt, membt, ubt, mbt, tails)


def _tc_finish(partials, bu2d, bm2d):
    def body(p_ref, bu_ref, bm_ref, o_ref):
        s = jnp.sum(p_ref[...])
        o_ref[...] = jax.nn.sigmoid(bu_ref[...] + bm_ref[...] + s)

    return pl.pallas_call(
        body,
        out_shape=jax.ShapeDtypeStruct(bu2d.shape, jnp.float32),
    )(partials, bu2d, bm2d)


def kernel(inputs, user_emb, user_bias_tab, movie_emb, movie_bias_tab):
    uidx = inputs[:, 0]
    midx = inputs[:, 1]
    tails = jnp.concatenate([
        user_emb[VMAIN:].reshape(-1),
        movie_emb[VMAIN:].reshape(-1),
        user_bias_tab[VMAIN:, 0],
        movie_bias_tab[VMAIN:, 0],
        jnp.zeros((T_LEN - T_MB - VTAIL,), jnp.float32),
    ])
    partials, bu, bm = _sc_main(uidx, midx, user_emb.T, movie_emb.T,
                                user_bias_tab.T, movie_bias_tab.T, tails)
    out2d = _tc_finish(partials, bu.reshape(128, 128), bm.reshape(128, 128))
    return out2d.reshape(B, 1)


# speedup vs baseline: 2.0169x; 1.3312x over previous
import functools

import jax
import jax.numpy as jnp
from jax import lax
from jax.experimental import pallas as pl
from jax.experimental.pallas import tpu as pltpu
from jax.experimental.pallas import tpu_sc as plsc

B = 16384
E = 64
V = 100000
L = 16
NC = 2
NS = 16
NW = NC * NS
VMAIN = 99968
VTAIL = V - VMAIN
VPAD = VMAIN + 128
S0 = 50048
S1 = VMAIN - S0
IQ = B // 4
T_U, T_M, T_UB, T_MB, T_LEN = 0, 2048, 4096, 4128, 5120


def _sc_main(uidx, midx, uembt, membt, ubt, mbt, tails):
    @functools.partial(
        pl.kernel,
        out_type=[
            jax.ShapeDtypeStruct((NW, 128), jnp.float32),
            jax.ShapeDtypeStruct((B,), jnp.float32),
            jax.ShapeDtypeStruct((B,), jnp.float32),
        ],
        mesh=plsc.VectorSubcoreMesh(core_axis_name="c", subcore_axis_name="s"),
        compiler_params=pltpu.CompilerParams(needs_layout_passes=False),
        scratch_types=[
            pltpu.VMEM((VPAD,), jnp.float32),
            pltpu.VMEM((B,), jnp.float32),
            pltpu.VMEM((IQ,), jnp.int32),
            pltpu.VMEM((IQ,), jnp.int32),
            pltpu.VMEM((T_LEN,), jnp.float32),
            pltpu.VMEM((128,), jnp.float32),
            pltpu.SemaphoreType.DMA,
            pltpu.SemaphoreType.DMA,
            pltpu.SemaphoreType.DMA,
        ],
    )
    def k(uidx_hbm, midx_hbm, uembt_hbm, membt_hbm, ubt_hbm, mbt_hbm, tails_hbm,
          partials_out, bu_out, bm_out,
          rowv, uvec, idxa, idxb, tails_v, o128, sem0, sem1, semi):
        wid = lax.axis_index("s") * NC + lax.axis_index("c")
        lanes = jnp.arange(L, dtype=jnp.int32)
        pltpu.sync_copy(tails_hbm, tails_v)
        ibufs = (idxa, idxb)

        def stage_row(src2d, row, tbase, is_bias):
            cp0 = pltpu.async_copy(src2d.at[row, pl.ds(0, S0)],
                                   rowv.at[pl.ds(0, S0)], sem0)
            cp1 = pltpu.async_copy(src2d.at[row, pl.ds(S0, S1)],
                                   rowv.at[pl.ds(S0, S1)], sem1)
            cp0.wait()
            cp1.wait()
            for c in range(VTAIL // L):
                rel = lanes + c * L
                fidx = tbase + (rel if is_bias else rel * E + row)
                rowv[pl.ds(VMAIN + c * L, L)] = plsc.load_gather(tails_v, [fidx])

        def extract_quarter(q, idxv, qout=None):
            qo = q if qout is None else qout
            def body(i, _):
                for t in range(8):
                    off = i * 128 + t * L
                    ic = idxv[pl.ds(off, L)]
                    uvec[pl.ds(qo * IQ + off, L)] = plsc.load_gather(rowv, [ic])
                return 0
            lax.fori_loop(0, IQ // 128, body, 0)

        def accum_quarter(q, idxv, acc0):
            def body(i, acc):
                for t in range(8):
                    off = i * 128 + t * L
                    ic = idxv[pl.ds(off, L)]
                    g = plsc.load_gather(rowv, [ic])
                    acc = acc + g * uvec[pl.ds(q * IQ + off, L)]
                return acc
            return lax.fori_loop(0, IQ // 128, body, acc0)

        def extract_all(idx_hbm):
            cp = pltpu.async_copy(idx_hbm.at[pl.ds(0, IQ)], ibufs[0], semi)
            for q in range(4):
                cp.wait()
                if q < 3:
                    cp = pltpu.async_copy(
                        idx_hbm.at[pl.ds((q + 1) * IQ, IQ)], ibufs[(q + 1) % 2],
                        semi)
                extract_quarter(q, ibufs[q % 2])

        def accum_all(idx_hbm, acc):
            cp = pltpu.async_copy(idx_hbm.at[pl.ds(0, IQ)], ibufs[0], semi)
            for q in range(4):
                cp.wait()
                if q < 3:
                    cp = pltpu.async_copy(
                        idx_hbm.at[pl.ds((q + 1) * IQ, IQ)], ibufs[(q + 1) % 2],
                        semi)
                acc = accum_quarter(q, ibufs[q % 2], acc)
            return acc

        @pl.when(wid < 4)
        def _():
            stage_row(ubt_hbm, 0, T_UB, True)
            q = wid
            pltpu.sync_copy(uidx_hbm.at[pl.ds(q * IQ, IQ)], idxa)
            extract_quarter(q, idxa, qout=0)
            pltpu.sync_copy(uvec.at[pl.ds(0, IQ)], bu_out.at[pl.ds(q * IQ, IQ)])

        @pl.when((wid >= 4) & (wid < 8))
        def _():
            stage_row(mbt_hbm, 0, T_MB, True)
            q = wid - 4
            pltpu.sync_copy(midx_hbm.at[pl.ds(q * IQ, IQ)], idxa)
            extract_quarter(q, idxa, qout=0)
            pltpu.sync_copy(uvec.at[pl.ds(0, IQ)], bm_out.at[pl.ds(q * IQ, IQ)])

        acc = jnp.zeros((L,), jnp.float32)
        for j in range(2):
            e = wid * 2 + j
            stage_row(uembt_hbm, e, T_U, False)
            extract_all(uidx_hbm)
            stage_row(membt_hbm, e, T_M, False)
            acc = accum_all(midx_hbm, acc)

        o128[pl.ds(0, L)] = acc
        zeros = jnp.zeros((L,), jnp.float32)
        for c in range(1, 128 // L):
            o128[pl.ds(c * L, L)] = zeros
        pltpu.sync_copy(o128, partials_out.at[wid])

    return k(uidx, midx, uembt, membt, ubt, mbt, tails)


def _tc_finish(partials, bu2d, bm2d):
    def body(p_ref, bu_ref, bm_ref, o_ref):
        s = jnp.sum(p_ref[...])
        o_ref[...] = jax.nn.sigmoid(bu_ref[...] + bm_ref[...] + s)

    return pl.pallas_call(
        body,
        out_shape=jax.ShapeDtypeStruct(bu2d.shape, jnp.float32),
    )(partials, bu2d, bm2d)


def kernel(inputs, user_emb, user_bias_tab, movie_emb, movie_bias_tab):
    uidx = inputs[:, 0]
    midx = inputs[:, 1]
    tails = jnp.concatenate([
        user_emb[VMAIN:].reshape(-1),
        movie_emb[VMAIN:].reshape(-1),
        user_bias_tab[VMAIN:, 0],
        movie_bias_tab[VMAIN:, 0],
        jnp.zeros((T_LEN - T_MB - VTAIL,), jnp.float32),
    ])
    partials, bu, bm = _sc_main(uidx, midx, user_emb.T, movie_emb.T,
                                user_bias_tab.T, movie_bias_tab.T, tails)
    out2d = _tc_finish(partials, bu.reshape(128, 128), bm.reshape(128, 128))
    return out2d.reshape(B, 1)
